# Initial kernel scaffold; baseline (speedup 1.0000x reference)
import jax
import jax.numpy as jnp
from jax.experimental import pallas as pl


def kernel(x, emb_table):
    raise NotImplementedError("write your pallas kernel here")


# speedup vs baseline: 1.7204x; 1.7204x over previous
import jax
import jax.numpy as jnp
from jax.experimental import pallas as pl

_BT = 512


def _body(x_ref, e_ref, o_ref):
    o_ref[...] = x_ref[...] + e_ref[...][None, :, :]


def kernel(x, emb_table):
    B, T, D = x.shape
    nT = T // _BT
    return pl.pallas_call(
        _body,
        grid=(nT,),
        in_specs=[
            pl.BlockSpec((B, _BT, D), lambda t: (0, t, 0)),
            pl.BlockSpec((_BT, D), lambda t: (t, 0)),
        ],
        out_specs=pl.BlockSpec((B, _BT, D), lambda t: (0, t, 0)),
        out_shape=jax.ShapeDtypeStruct(x.shape, x.dtype),
    )(x, emb_table)
